# baseline (speedup 1.0000x reference)
import functools

import jax
import jax.numpy as jnp
from jax import lax
from jax.experimental import pallas as pl
from jax.experimental.pallas import tpu as pltpu
from jax.experimental.pallas import tpu_sc as plsc

_N = 10000
_E = 320000
_EH0 = 166400
_EH1 = _E - _EH0
_D = 128
_H = 300
_L = 100
_G = 3
_LP = 128
_NW = 32
_NP = 10240
_ROWS_PER_TILE = _NP // 16
_ZR = 64

_CB = 400
_CG = 80
_CS = 80


def _worker_id():
    return lax.axis_index("s") * 2 + lax.axis_index("c")


def _make_gather(eh, dp):
    perw = eh // _NW
    mesh = plsc.VectorSubcoreMesh(core_axis_name="c", subcore_axis_name="s")

    @functools.partial(
        pl.kernel,
        out_type=(jax.ShapeDtypeStruct((eh, dp), jnp.float32),
                  jax.ShapeDtypeStruct((eh, dp), jnp.float32)),
        mesh=mesh,
        scratch_types=[
            pltpu.VMEM((perw,), jnp.int32),
            pltpu.VMEM((perw,), jnp.int32),
            pltpu.VMEM((_CB, dp), jnp.float32),
            pltpu.VMEM((_CB, dp), jnp.float32),
            pltpu.SemaphoreType.DMA,
            pltpu.SemaphoreType.DMA,
            pltpu.SemaphoreType.DMA,
            pltpu.SemaphoreType.DMA,
        ],
    )
    def gk(table, dsti, srci, outd, outs, div, siv, drv, srv, sgd, sgs, swd, sws):
        base = _worker_id() * perw
        pltpu.sync_copy(dsti.at[pl.ds(base, perw)], div)
        pltpu.sync_copy(srci.at[pl.ds(base, perw)], siv)
        nsub = _CB // _CG

        def body(i, carry):
            @pl.when(i > 0)
            def _():
                pltpu.make_async_copy(drv, outd.at[pl.ds(base, _CB)], swd).wait()
                pltpu.make_async_copy(srv, outs.at[pl.ds(base, _CB)], sws).wait()

            cps = []
            for buf, ivec, sem in ((drv, div, sgd), (srv, siv, sgs)):
                for k in range(nsub):
                    o = k * _CG
                    cps.append(pltpu.async_copy(
                        table.at[ivec.at[pl.ds(i * _CB + o, _CG)]],
                        buf.at[pl.ds(o, _CG)], sem))
            for cp in cps[:nsub]:
                cp.wait()
            pltpu.async_copy(drv, outd.at[pl.ds(base + i * _CB, _CB)], swd)
            for cp in cps[nsub:]:
                cp.wait()
            pltpu.async_copy(srv, outs.at[pl.ds(base + i * _CB, _CB)], sws)
            return carry

        lax.fori_loop(0, perw // _CB, body, 0)
        pltpu.make_async_copy(drv, outd.at[pl.ds(base, _CB)], swd).wait()
        pltpu.make_async_copy(srv, outs.at[pl.ds(base, _CB)], sws).wait()

    return gk


_gathers = (_make_gather(_EH0, _LP), _make_gather(_EH1, _LP))


def _make_scatter(eh):
    perw = eh // _NW
    mesh = plsc.VectorSubcoreMesh(core_axis_name="c", subcore_axis_name="s")

    @functools.partial(
        pl.kernel,
        out_type=jax.ShapeDtypeStruct((2 * _NP, _LP), jnp.float32),
        mesh=mesh,
        scratch_types=[
            pltpu.VMEM((_CS,), jnp.int32),
            pltpu.VMEM((_CS,), jnp.int32),
            pltpu.VMEM((_CS, _LP), jnp.float32),
            pltpu.VMEM((_CS, _LP), jnp.float32),
            pltpu.VMEM((_ZR, _LP), jnp.float32),
            pltpu.VMEM_SHARED((_NP, _LP), jnp.float32),
            pltpu.SemaphoreType.DMA,
            pltpu.SemaphoreType.DMA,
        ],
    )
    def sk(m_hbm, dsti, parts, ib0, ib1, mb0, mb1, zv, acc, sd0, sd1):
        c_ax = lax.axis_index("c")
        s_ax = lax.axis_index("s")
        base = (s_ax * 2 + c_ax) * perw
        ibufs = (ib0, ib1)
        mbufs = (mb0, mb1)
        sds = (sd0, sd1)
        zero16 = jnp.zeros((16,), jnp.float32)

        def zbody(i, carry):
            r = i // (_LP // 16)
            j = i % (_LP // 16)
            zv[r, pl.ds(j * 16, 16)] = zero16
            return carry

        lax.fori_loop(0, _ZR * (_LP // 16), zbody, 0)
        for k in range(_ROWS_PER_TILE // _ZR):
            pltpu.sync_copy(zv, acc.at[pl.ds(s_ax * _ROWS_PER_TILE + k * _ZR, _ZR)])
        plsc.subcore_barrier()

        nchunk = perw // _CS

        def load(p, ci):
            off = base + ci * _CS
            pltpu.async_copy(m_hbm.at[pl.ds(off, _CS)], mbufs[p], sds[p])
            pltpu.async_copy(dsti.at[pl.ds(off, _CS)], ibufs[p], sds[p])

        def drain(p, ci):
            off = base + ci * _CS
            pltpu.make_async_copy(m_hbm.at[pl.ds(off, _CS)], mbufs[p], sds[p]).wait()
            pltpu.make_async_copy(dsti.at[pl.ds(off, _CS)], ibufs[p], sds[p]).wait()

        def scadd(p):
            pltpu.sync_copy(mbufs[p], acc.at[ibufs[p]], add=True)

        npairs = nchunk // 2
        load(0, 0)
        load(1, 1)

        def body(it, carry):
            c = 2 * it
            drain(0, c)
            scadd(0)

            @pl.when(c + 2 < nchunk)
            def _():
                load(0, c + 2)

            drain(1, c + 1)
            scadd(1)

            @pl.when(c + 3 < nchunk)
            def _():
                load(1, c + 3)

            return carry

        lax.fori_loop(0, npairs, body, 0)
        if nchunk % 2:
            drain(0, nchunk - 1)
            scadd(0)
        plsc.subcore_barrier()
        for k in range(_ROWS_PER_TILE // _ZR):
            row0 = s_ax * _ROWS_PER_TILE + k * _ZR
            pltpu.sync_copy(acc.at[pl.ds(row0, _ZR)], zv)
            pltpu.sync_copy(zv, parts.at[pl.ds(c_ax * _NP + row0, _ZR)])

    return sk


_scatters = (_make_scatter(_EH0), _make_scatter(_EH1))


def _edge_mlp(gd, gs, w1d, w1s, b1, w2, b2, w3, b3):
    be = 2560
    eh = gd.shape[0]
    din = gd.shape[1]

    def body(gd_ref, gs_ref, w1d_r, w1s_r, b1_r, w2_r, b2_r, w3_r, b3_r, o_ref):
        h = jnp.dot(gd_ref[...], w1d_r[...], preferred_element_type=jnp.float32, precision=lax.Precision.HIGHEST)
        h += jnp.dot(gs_ref[...], w1s_r[...], preferred_element_type=jnp.float32, precision=lax.Precision.HIGHEST)
        h = jnp.maximum(h + b1_r[...], 0.0)
        h = jnp.dot(h, w2_r[...], preferred_element_type=jnp.float32, precision=lax.Precision.HIGHEST) + b2_r[...]
        h = jnp.maximum(h, 0.0)
        o_ref[...] = jnp.dot(h, w3_r[...], preferred_element_type=jnp.float32, precision=lax.Precision.HIGHEST) + b3_r[...]

    full = lambda a: pl.BlockSpec(a.shape, lambda i: (0,) * a.ndim)
    return pl.pallas_call(
        body,
        grid=(eh // be,),
        in_specs=[
            pl.BlockSpec((be, din), lambda i: (i, 0)),
            pl.BlockSpec((be, din), lambda i: (i, 0)),
            full(w1d), full(w1s), full(b1), full(w2), full(b2), full(w3), full(b3),
        ],
        out_specs=pl.BlockSpec((be, _LP), lambda i: (i, 0)),
        out_shape=jax.ShapeDtypeStruct((eh, _LP), jnp.float32),
        compiler_params=pltpu.CompilerParams(
            dimension_semantics=("arbitrary",)),
    )(gd, gs, w1d, w1s, b1, w2, b2, w3, b3)


def _node_mean(parts_a, parts_b):
    bn = 2048

    def body(pa_ref, pb_ref, o_ref):
        sacc = pa_ref[0] + pa_ref[1] + pb_ref[0] + pb_ref[1]
        cnt = jnp.maximum(sacc[:, 100:101], 1.0)
        h = jnp.maximum(sacc / cnt, 0.0)
        col = lax.broadcasted_iota(jnp.int32, (bn, _LP), 1)
        o_ref[...] = jnp.where(col < _L, h, 0.0)

    spec = pl.BlockSpec((2, bn, _LP), lambda i: (0, i, 0))
    return pl.pallas_call(
        body,
        grid=(_NP // bn,),
        in_specs=[spec, spec],
        out_specs=pl.BlockSpec((bn, _LP), lambda i: (i, 0)),
        out_shape=jax.ShapeDtypeStruct((_NP, _LP), jnp.float32),
        compiler_params=pltpu.CompilerParams(
            dimension_semantics=("arbitrary",)),
    )(parts_a.reshape(2, _NP, _LP), parts_b.reshape(2, _NP, _LP))


def _head(parts_a, parts_b, u, fw1h, fw1u, fb1, fw2, fb2, fw3, fb3):
    def body(pa_ref, pb_ref, u_ref, w1h_r, w1u_r, b1_r, w2_r, b2_r, w3_r,
             b3_r, o_ref):
        sacc = pa_ref[0] + pa_ref[1] + pb_ref[0] + pb_ref[1]
        cnt = jnp.maximum(sacc[:, 100:101], 1.0)
        h = jnp.maximum(sacc / cnt, 0.0)
        pooled = jnp.sum(h, axis=0, keepdims=True) * (1.0 / _N)
        z = jnp.dot(pooled, w1h_r[...], preferred_element_type=jnp.float32, precision=lax.Precision.HIGHEST)
        z += jnp.dot(u_ref[...], w1u_r[...], preferred_element_type=jnp.float32, precision=lax.Precision.HIGHEST)
        z = jnp.maximum(z + b1_r[...], 0.0)
        z = jnp.maximum(
            jnp.dot(z, w2_r[...], preferred_element_type=jnp.float32, precision=lax.Precision.HIGHEST) + b2_r[...], 0.0)
        o_ref[...] = jnp.dot(z, w3_r[...], preferred_element_type=jnp.float32, precision=lax.Precision.HIGHEST) + b3_r[...]

    return pl.pallas_call(
        body,
        out_shape=jax.ShapeDtypeStruct((1, 1), jnp.float32),
    )(parts_a.reshape(2, _NP, _LP), parts_b.reshape(2, _NP, _LP),
      u, fw1h, fw1u, fb1, fw2, fb2, fw3, fb3)


def kernel(x, edge_index, batch, u,
           l0_W1, l0_b1, l0_W2, l0_b2, l0_W3, l0_b3,
           l1_W1, l1_b1, l1_W2, l1_b2, l1_W3, l1_b3,
           f_W1, f_b1, f_W2, f_b2, f_W3, f_b3):
    dst = edge_index[1]
    src = edge_index[0]
    dsts = (dst[:_EH0], dst[_EH0:])
    srcs = (src[:_EH0], src[_EH0:])

    w1d0 = l0_W1[:_D] - l0_W1[_D:]
    w1s0 = l0_W1[_D:]
    w1d1 = jnp.pad(l1_W1[:_L] - l1_W1[_L:], ((0, _LP - _L), (0, 0)))
    w1s1 = jnp.pad(l1_W1[_L:], ((0, _LP - _L), (0, 0)))
    onehot = jnp.zeros((_LP,), jnp.float32).at[_L].set(1.0)

    def pad3(w3, b3):
        w3p = jnp.pad(w3, ((0, 0), (0, _LP - _L)))
        b3p = jnp.pad(b3, (0, _LP - _L)) + onehot
        return w3p, b3p.reshape(1, _LP)

    w3p0, b3p0 = pad3(l0_W3, l0_b3)
    w3p1, b3p1 = pad3(l1_W3, l1_b3)
    fw1h = jnp.pad(f_W1[:_L], ((0, _LP - _L), (0, 0)))
    fw1u = f_W1[_L:]
    xp = jnp.concatenate([x, jnp.zeros((_N, _LP - _D), x.dtype)], axis=1) \
        if _D != _LP else x

    def layer(tab, w1d, w1s, b1, w2, b2, w3p, b3p):
        parts = []
        for half in (0, 1):
            gd, gs = _gathers[half](tab, dsts[half], srcs[half])
            m = _edge_mlp(gd, gs, w1d, w1s, b1, w2, b2, w3p, b3p)
            parts.append(_scatters[half](m, dsts[half]))
        return parts

    pa0, pb0 = layer(xp, w1d0, w1s0, l0_b1.reshape(1, _H),
                     l0_W2, l0_b2.reshape(1, _H), w3p0, b3p0)
    h0 = _node_mean(pa0, pb0)
    pa1, pb1 = layer(h0, w1d1, w1s1, l1_b1.reshape(1, _H),
                     l1_W2, l1_b2.reshape(1, _H), w3p1, b3p1)
    return _head(pa1, pb1, u, fw1h, fw1u, f_b1.reshape(1, _L),
                 f_W2, f_b2.reshape(1, _L), f_W3, f_b3.reshape(1, 1))


# speedup vs baseline: 2.4888x; 2.4888x over previous
import functools

import jax
import jax.numpy as jnp
from jax import lax
from jax.experimental import pallas as pl
from jax.experimental.pallas import tpu as pltpu
from jax.experimental.pallas import tpu_sc as plsc

_N = 10000
_E = 320000
_EH0 = 166400
_EH1 = _E - _EH0
_D = 128
_H = 300
_L = 100
_G = 3
_LP = 128
_NW = 32
_NP = 10240
_ROWS_PER_TILE = _NP // 16
_ZR = 64

_CB = 400
_CG = 80
_CS = 80


def _worker_id():
    return lax.axis_index("s") * 2 + lax.axis_index("c")


def _make_gather(eh, dp):
    perw = eh // _NW
    mesh = plsc.VectorSubcoreMesh(core_axis_name="c", subcore_axis_name="s")

    @functools.partial(
        pl.kernel,
        out_type=(jax.ShapeDtypeStruct((eh, dp), jnp.float32),
                  jax.ShapeDtypeStruct((eh, dp), jnp.float32)),
        mesh=mesh,
        scratch_types=[
            pltpu.VMEM((perw,), jnp.int32),
            pltpu.VMEM((perw,), jnp.int32),
            pltpu.VMEM((_CB, dp), jnp.float32),
            pltpu.VMEM((_CB, dp), jnp.float32),
            pltpu.SemaphoreType.DMA,
            pltpu.SemaphoreType.DMA,
            pltpu.SemaphoreType.DMA,
            pltpu.SemaphoreType.DMA,
        ],
    )
    def gk(table, dsti, srci, outd, outs, div, siv, drv, srv, sgd, sgs, swd, sws):
        base = _worker_id() * perw
        pltpu.sync_copy(dsti.at[pl.ds(base, perw)], div)
        pltpu.sync_copy(srci.at[pl.ds(base, perw)], siv)
        nsub = _CB // _CG

        def body(i, carry):
            @pl.when(i > 0)
            def _():
                pltpu.make_async_copy(drv, outd.at[pl.ds(base, _CB)], swd).wait()
                pltpu.make_async_copy(srv, outs.at[pl.ds(base, _CB)], sws).wait()

            cps = []
            for buf, ivec, sem in ((drv, div, sgd), (srv, siv, sgs)):
                for k in range(nsub):
                    o = k * _CG
                    cps.append(pltpu.async_copy(
                        table.at[ivec.at[pl.ds(i * _CB + o, _CG)]],
                        buf.at[pl.ds(o, _CG)], sem))
            for cp in cps[:nsub]:
                cp.wait()
            pltpu.async_copy(drv, outd.at[pl.ds(base + i * _CB, _CB)], swd)
            for cp in cps[nsub:]:
                cp.wait()
            pltpu.async_copy(srv, outs.at[pl.ds(base + i * _CB, _CB)], sws)
            return carry

        lax.fori_loop(0, perw // _CB, body, 0)
        pltpu.make_async_copy(drv, outd.at[pl.ds(base, _CB)], swd).wait()
        pltpu.make_async_copy(srv, outs.at[pl.ds(base, _CB)], sws).wait()

    return gk


_gathers = (_make_gather(_EH0, _LP), _make_gather(_EH1, _LP))


def _make_scatter(eh):
    perw = eh // _NW
    mesh = plsc.VectorSubcoreMesh(core_axis_name="c", subcore_axis_name="s")

    @functools.partial(
        pl.kernel,
        out_type=jax.ShapeDtypeStruct((2 * _NP, _LP), jnp.float32),
        mesh=mesh,
        scratch_types=[
            pltpu.VMEM((_CS,), jnp.int32),
            pltpu.VMEM((_CS,), jnp.int32),
            pltpu.VMEM((_CS, _LP), jnp.float32),
            pltpu.VMEM((_CS, _LP), jnp.float32),
            pltpu.VMEM((_ZR, _LP), jnp.float32),
            pltpu.VMEM_SHARED((_NP, _LP), jnp.float32),
            pltpu.SemaphoreType.DMA,
            pltpu.SemaphoreType.DMA,
        ],
    )
    def sk(m_hbm, dsti, parts, ib0, ib1, mb0, mb1, zv, acc, sd0, sd1):
        c_ax = lax.axis_index("c")
        s_ax = lax.axis_index("s")
        base = (s_ax * 2 + c_ax) * perw
        ibufs = (ib0, ib1)
        mbufs = (mb0, mb1)
        sds = (sd0, sd1)
        zero16 = jnp.zeros((16,), jnp.float32)

        def zbody(i, carry):
            r = i // (_LP // 16)
            j = i % (_LP // 16)
            zv[r, pl.ds(j * 16, 16)] = zero16
            return carry

        lax.fori_loop(0, _ZR * (_LP // 16), zbody, 0)
        for k in range(_ROWS_PER_TILE // _ZR):
            pltpu.sync_copy(zv, acc.at[pl.ds(s_ax * _ROWS_PER_TILE + k * _ZR, _ZR)])
        plsc.subcore_barrier()

        nchunk = perw // _CS

        def load(p, ci):
            off = base + ci * _CS
            pltpu.async_copy(m_hbm.at[pl.ds(off, _CS)], mbufs[p], sds[p])
            pltpu.async_copy(dsti.at[pl.ds(off, _CS)], ibufs[p], sds[p])

        def drain(p, ci):
            off = base + ci * _CS
            pltpu.make_async_copy(m_hbm.at[pl.ds(off, _CS)], mbufs[p], sds[p]).wait()
            pltpu.make_async_copy(dsti.at[pl.ds(off, _CS)], ibufs[p], sds[p]).wait()

        def scadd(p):
            pltpu.sync_copy(mbufs[p], acc.at[ibufs[p]], add=True)

        npairs = nchunk // 2
        load(0, 0)
        load(1, 1)

        def body(it, carry):
            c = 2 * it
            drain(0, c)
            scadd(0)

            @pl.when(c + 2 < nchunk)
            def _():
                load(0, c + 2)

            drain(1, c + 1)
            scadd(1)

            @pl.when(c + 3 < nchunk)
            def _():
                load(1, c + 3)

            return carry

        lax.fori_loop(0, npairs, body, 0)
        if nchunk % 2:
            drain(0, nchunk - 1)
            scadd(0)
        plsc.subcore_barrier()
        for k in range(_ROWS_PER_TILE // _ZR):
            row0 = s_ax * _ROWS_PER_TILE + k * _ZR
            pltpu.sync_copy(acc.at[pl.ds(row0, _ZR)], zv)
            pltpu.sync_copy(zv, parts.at[pl.ds(c_ax * _NP + row0, _ZR)])

    return sk


_scatters = (_make_scatter(_EH0), _make_scatter(_EH1))


def _edge_mlp(gd, gs, w1d, w1s, b1, w2, b2, w3, b3):
    be = 2560
    eh = gd.shape[0]
    din = gd.shape[1]

    bf = jnp.bfloat16

    def dot2(a, w):
        hi = jnp.dot(a, w[0][...], preferred_element_type=jnp.float32)
        lo = jnp.dot(a, w[1][...], preferred_element_type=jnp.float32)
        return hi + lo

    def body(gd_ref, gs_ref, w1dh, w1dl, w1sh, w1sl, b1_r, w2h, w2l, b2_r,
             w3h, w3l, b3_r, o_ref):
        h = dot2(gd_ref[...].astype(bf), (w1dh, w1dl))
        h += dot2(gs_ref[...].astype(bf), (w1sh, w1sl))
        h = jnp.maximum(h + b1_r[...], 0.0)
        h = jnp.maximum(dot2(h.astype(bf), (w2h, w2l)) + b2_r[...], 0.0)
        o_ref[...] = dot2(h.astype(bf), (w3h, w3l)) + b3_r[...]

    full = lambda a: pl.BlockSpec(a.shape, lambda i: (0,) * a.ndim)
    wargs = (w1d[0], w1d[1], w1s[0], w1s[1], b1, w2[0], w2[1], b2,
             w3[0], w3[1], b3)
    return pl.pallas_call(
        body,
        grid=(eh // be,),
        in_specs=[
            pl.BlockSpec((be, din), lambda i: (i, 0)),
            pl.BlockSpec((be, din), lambda i: (i, 0)),
        ] + [full(a) for a in wargs],
        out_specs=pl.BlockSpec((be, _LP), lambda i: (i, 0)),
        out_shape=jax.ShapeDtypeStruct((eh, _LP), jnp.float32),
        compiler_params=pltpu.CompilerParams(
            dimension_semantics=("arbitrary",)),
    )(gd, gs, *wargs)


def _node_mean(parts_a, parts_b):
    bn = 2048

    def body(pa_ref, pb_ref, o_ref):
        sacc = pa_ref[0] + pa_ref[1] + pb_ref[0] + pb_ref[1]
        cnt = jnp.maximum(sacc[:, 100:101], 1.0)
        h = jnp.maximum(sacc / cnt, 0.0)
        col = lax.broadcasted_iota(jnp.int32, (bn, _LP), 1)
        o_ref[...] = jnp.where(col < _L, h, 0.0)

    spec = pl.BlockSpec((2, bn, _LP), lambda i: (0, i, 0))
    return pl.pallas_call(
        body,
        grid=(_NP // bn,),
        in_specs=[spec, spec],
        out_specs=pl.BlockSpec((bn, _LP), lambda i: (i, 0)),
        out_shape=jax.ShapeDtypeStruct((_NP, _LP), jnp.float32),
        compiler_params=pltpu.CompilerParams(
            dimension_semantics=("arbitrary",)),
    )(parts_a.reshape(2, _NP, _LP), parts_b.reshape(2, _NP, _LP))


def _head(parts_a, parts_b, u, fw1h, fw1u, fb1, fw2, fb2, fw3, fb3):
    def body(pa_ref, pb_ref, u_ref, w1h_r, w1u_r, b1_r, w2_r, b2_r, w3_r,
             b3_r, o_ref):
        sacc = pa_ref[0] + pa_ref[1] + pb_ref[0] + pb_ref[1]
        cnt = jnp.maximum(sacc[:, 100:101], 1.0)
        h = jnp.maximum(sacc / cnt, 0.0)
        pooled = jnp.sum(h, axis=0, keepdims=True) * (1.0 / _N)
        z = jnp.dot(pooled, w1h_r[...], preferred_element_type=jnp.float32, precision=lax.Precision.HIGHEST)
        z += jnp.dot(u_ref[...], w1u_r[...], preferred_element_type=jnp.float32, precision=lax.Precision.HIGHEST)
        z = jnp.maximum(z + b1_r[...], 0.0)
        z = jnp.maximum(
            jnp.dot(z, w2_r[...], preferred_element_type=jnp.float32, precision=lax.Precision.HIGHEST) + b2_r[...], 0.0)
        o_ref[...] = jnp.dot(z, w3_r[...], preferred_element_type=jnp.float32, precision=lax.Precision.HIGHEST) + b3_r[...]

    return pl.pallas_call(
        body,
        out_shape=jax.ShapeDtypeStruct((1, 1), jnp.float32),
    )(parts_a.reshape(2, _NP, _LP), parts_b.reshape(2, _NP, _LP),
      u, fw1h, fw1u, fb1, fw2, fb2, fw3, fb3)


def kernel(x, edge_index, batch, u,
           l0_W1, l0_b1, l0_W2, l0_b2, l0_W3, l0_b3,
           l1_W1, l1_b1, l1_W2, l1_b2, l1_W3, l1_b3,
           f_W1, f_b1, f_W2, f_b2, f_W3, f_b3):
    dst = edge_index[1]
    src = edge_index[0]
    dsts = (dst[:_EH0], dst[_EH0:])
    srcs = (src[:_EH0], src[_EH0:])

    w1d0 = l0_W1[:_D] - l0_W1[_D:]
    w1s0 = l0_W1[_D:]
    w1d1 = jnp.pad(l1_W1[:_L] - l1_W1[_L:], ((0, _LP - _L), (0, 0)))
    w1s1 = jnp.pad(l1_W1[_L:], ((0, _LP - _L), (0, 0)))
    onehot = jnp.zeros((_LP,), jnp.float32).at[_L].set(1.0)

    def pad3(w3, b3):
        w3p = jnp.pad(w3, ((0, 0), (0, _LP - _L)))
        b3p = jnp.pad(b3, (0, _LP - _L)) + onehot
        return w3p, b3p.reshape(1, _LP)

    w3p0, b3p0 = pad3(l0_W3, l0_b3)
    w3p1, b3p1 = pad3(l1_W3, l1_b3)
    fw1h = jnp.pad(f_W1[:_L], ((0, _LP - _L), (0, 0)))
    fw1u = f_W1[_L:]
    xp = jnp.concatenate([x, jnp.zeros((_N, _LP - _D), x.dtype)], axis=1) \
        if _D != _LP else x

    def split(w):
        bf = jnp.bfloat16
        hi = w.astype(bf)
        lo = (w - hi.astype(jnp.float32)).astype(bf)
        return hi, lo

    def layer(tab, w1d, w1s, b1, w2, b2, w3p, b3p):
        parts = []
        for half in (0, 1):
            gd, gs = _gathers[half](tab, dsts[half], srcs[half])
            m = _edge_mlp(gd, gs, split(w1d), split(w1s), b1,
                          split(w2), b2, split(w3p), b3p)
            parts.append(_scatters[half](m, dsts[half]))
        return parts

    pa0, pb0 = layer(xp, w1d0, w1s0, l0_b1.reshape(1, _H),
                     l0_W2, l0_b2.reshape(1, _H), w3p0, b3p0)
    h0 = _node_mean(pa0, pb0)
    pa1, pb1 = layer(h0, w1d1, w1s1, l1_b1.reshape(1, _H),
                     l1_W2, l1_b2.reshape(1, _H), w3p1, b3p1)
    return _head(pa1, pb1, u, fw1h, fw1u, f_b1.reshape(1, _L),
                 f_W2, f_b2.reshape(1, _L), f_W3, f_b3.reshape(1, 1))


# speedup vs baseline: 2.6067x; 1.0474x over previous
import functools

import jax
import jax.numpy as jnp
from jax import lax
from jax.experimental import pallas as pl
from jax.experimental.pallas import tpu as pltpu
from jax.experimental.pallas import tpu_sc as plsc

_N = 10000
_E = 320000
_CHUNKS = (64000, 128000, 128000)
_D = 128
_H = 300
_L = 100
_G = 3
_LP = 128
_NW = 32
_NP = 10240
_ROWS_PER_TILE = _NP // 16
_ZR = 64

_CB = 400
_CG = 80
_CS = 80


def _worker_id():
    return lax.axis_index("s") * 2 + lax.axis_index("c")


def _make_gather(eh, dp):
    perw = eh // _NW
    mesh = plsc.VectorSubcoreMesh(core_axis_name="c", subcore_axis_name="s")

    @functools.partial(
        pl.kernel,
        out_type=(jax.ShapeDtypeStruct((eh, dp), jnp.float32),
                  jax.ShapeDtypeStruct((eh, dp), jnp.float32)),
        mesh=mesh,
        scratch_types=[
            pltpu.VMEM((perw,), jnp.int32),
            pltpu.VMEM((perw,), jnp.int32),
            pltpu.VMEM((_CB, dp), jnp.float32),
            pltpu.VMEM((_CB, dp), jnp.float32),
            pltpu.SemaphoreType.DMA,
            pltpu.SemaphoreType.DMA,
            pltpu.SemaphoreType.DMA,
            pltpu.SemaphoreType.DMA,
        ],
    )
    def gk(table, dsti, srci, outd, outs, div, siv, drv, srv, sgd, sgs, swd, sws):
        base = _worker_id() * perw
        pltpu.sync_copy(dsti.at[pl.ds(base, perw)], div)
        pltpu.sync_copy(srci.at[pl.ds(base, perw)], siv)
        nsub = _CB // _CG

        def body(i, carry):
            @pl.when(i > 0)
            def _():
                pltpu.make_async_copy(drv, outd.at[pl.ds(base, _CB)], swd).wait()
                pltpu.make_async_copy(srv, outs.at[pl.ds(base, _CB)], sws).wait()

            cps = []
            for buf, ivec, sem in ((drv, div, sgd), (srv, siv, sgs)):
                for k in range(nsub):
                    o = k * _CG
                    cps.append(pltpu.async_copy(
                        table.at[ivec.at[pl.ds(i * _CB + o, _CG)]],
                        buf.at[pl.ds(o, _CG)], sem))
            for cp in cps[:nsub]:
                cp.wait()
            pltpu.async_copy(drv, outd.at[pl.ds(base + i * _CB, _CB)], swd)
            for cp in cps[nsub:]:
                cp.wait()
            pltpu.async_copy(srv, outs.at[pl.ds(base + i * _CB, _CB)], sws)
            return carry

        lax.fori_loop(0, perw // _CB, body, 0)
        pltpu.make_async_copy(drv, outd.at[pl.ds(base, _CB)], swd).wait()
        pltpu.make_async_copy(srv, outs.at[pl.ds(base, _CB)], sws).wait()

    return gk


_gather_by_size = {eh: _make_gather(eh, _LP) for eh in set(_CHUNKS)}


def _make_scatter(eh):
    perw = eh // _NW
    mesh = plsc.VectorSubcoreMesh(core_axis_name="c", subcore_axis_name="s")

    @functools.partial(
        pl.kernel,
        out_type=jax.ShapeDtypeStruct((2 * _NP, _LP), jnp.float32),
        mesh=mesh,
        scratch_types=[
            pltpu.VMEM((_CS,), jnp.int32),
            pltpu.VMEM((_CS,), jnp.int32),
            pltpu.VMEM((_CS, _LP), jnp.float32),
            pltpu.VMEM((_CS, _LP), jnp.float32),
            pltpu.VMEM((_ZR, _LP), jnp.float32),
            pltpu.VMEM_SHARED((_NP, _LP), jnp.float32),
            pltpu.SemaphoreType.DMA,
            pltpu.SemaphoreType.DMA,
        ],
    )
    def sk(m_hbm, dsti, parts, ib0, ib1, mb0, mb1, zv, acc, sd0, sd1):
        c_ax = lax.axis_index("c")
        s_ax = lax.axis_index("s")
        base = (s_ax * 2 + c_ax) * perw
        ibufs = (ib0, ib1)
        mbufs = (mb0, mb1)
        sds = (sd0, sd1)
        zero16 = jnp.zeros((16,), jnp.float32)

        def zbody(i, carry):
            r = i // (_LP // 16)
            j = i % (_LP // 16)
            zv[r, pl.ds(j * 16, 16)] = zero16
            return carry

        lax.fori_loop(0, _ZR * (_LP // 16), zbody, 0)
        for k in range(_ROWS_PER_TILE // _ZR):
            pltpu.sync_copy(zv, acc.at[pl.ds(s_ax * _ROWS_PER_TILE + k * _ZR, _ZR)])
        plsc.subcore_barrier()

        nchunk = perw // _CS

        def load(p, ci):
            off = base + ci * _CS
            pltpu.async_copy(m_hbm.at[pl.ds(off, _CS)], mbufs[p], sds[p])
            pltpu.async_copy(dsti.at[pl.ds(off, _CS)], ibufs[p], sds[p])

        def drain(p, ci):
            off = base + ci * _CS
            pltpu.make_async_copy(m_hbm.at[pl.ds(off, _CS)], mbufs[p], sds[p]).wait()
            pltpu.make_async_copy(dsti.at[pl.ds(off, _CS)], ibufs[p], sds[p]).wait()

        def scadd(p):
            pltpu.sync_copy(mbufs[p], acc.at[ibufs[p]], add=True)

        npairs = nchunk // 2
        load(0, 0)
        load(1, 1)

        def body(it, carry):
            c = 2 * it
            drain(0, c)
            scadd(0)

            @pl.when(c + 2 < nchunk)
            def _():
                load(0, c + 2)

            drain(1, c + 1)
            scadd(1)

            @pl.when(c + 3 < nchunk)
            def _():
                load(1, c + 3)

            return carry

        lax.fori_loop(0, npairs, body, 0)
        if nchunk % 2:
            drain(0, nchunk - 1)
            scadd(0)
        plsc.subcore_barrier()
        for k in range(_ROWS_PER_TILE // _ZR):
            row0 = s_ax * _ROWS_PER_TILE + k * _ZR
            pltpu.sync_copy(acc.at[pl.ds(row0, _ZR)], zv)
            pltpu.sync_copy(zv, parts.at[pl.ds(c_ax * _NP + row0, _ZR)])

    return sk


_scatter_by_size = {eh: _make_scatter(eh) for eh in set(_CHUNKS)}


def _edge_mlp(gd, gs, w1d, w1s, b1, w2, b2, w3, b3):
    be = 2560
    eh = gd.shape[0]
    din = gd.shape[1]

    bf = jnp.bfloat16

    def dot2(a, w):
        hi = jnp.dot(a, w[0][...], preferred_element_type=jnp.float32)
        lo = jnp.dot(a, w[1][...], preferred_element_type=jnp.float32)
        return hi + lo

    def body(gd_ref, gs_ref, w1dh, w1dl, w1sh, w1sl, b1_r, w2h, w2l, b2_r,
             w3h, w3l, b3_r, o_ref):
        h = dot2(gd_ref[...].astype(bf), (w1dh, w1dl))
        h += dot2(gs_ref[...].astype(bf), (w1sh, w1sl))
        h = jnp.maximum(h + b1_r[...], 0.0)
        h = jnp.maximum(dot2(h.astype(bf), (w2h, w2l)) + b2_r[...], 0.0)
        o_ref[...] = dot2(h.astype(bf), (w3h, w3l)) + b3_r[...]

    full = lambda a: pl.BlockSpec(a.shape, lambda i: (0,) * a.ndim)
    wargs = (w1d[0], w1d[1], w1s[0], w1s[1], b1, w2[0], w2[1], b2,
             w3[0], w3[1], b3)
    return pl.pallas_call(
        body,
        grid=(eh // be,),
        in_specs=[
            pl.BlockSpec((be, din), lambda i: (i, 0)),
            pl.BlockSpec((be, din), lambda i: (i, 0)),
        ] + [full(a) for a in wargs],
        out_specs=pl.BlockSpec((be, _LP), lambda i: (i, 0)),
        out_shape=jax.ShapeDtypeStruct((eh, _LP), jnp.float32),
        compiler_params=pltpu.CompilerParams(
            dimension_semantics=("arbitrary",)),
    )(gd, gs, *wargs)


def _node_mean(parts):
    bn = 2048

    def body(*refs):
        o_ref = refs[-1]
        sacc = refs[0][0] + refs[0][1]
        for r in refs[1:-1]:
            sacc += r[0] + r[1]
        cnt = jnp.maximum(sacc[:, 100:101], 1.0)
        h = jnp.maximum(sacc / cnt, 0.0)
        col = lax.broadcasted_iota(jnp.int32, (bn, _LP), 1)
        o_ref[...] = jnp.where(col < _L, h, 0.0)

    spec = pl.BlockSpec((2, bn, _LP), lambda i: (0, i, 0))
    return pl.pallas_call(
        body,
        grid=(_NP // bn,),
        in_specs=[spec] * len(parts),
        out_specs=pl.BlockSpec((bn, _LP), lambda i: (i, 0)),
        out_shape=jax.ShapeDtypeStruct((_NP, _LP), jnp.float32),
        compiler_params=pltpu.CompilerParams(
            dimension_semantics=("arbitrary",)),
    )(*[p.reshape(2, _NP, _LP) for p in parts])


def _head(parts, u, fw1h, fw1u, fb1, fw2, fb2, fw3, fb3):
    nparts = len(parts)

    def body(*refs):
        prefs = refs[:nparts]
        u_ref, w1h_r, w1u_r, b1_r, w2_r, b2_r, w3_r, b3_r, o_ref = refs[nparts:]
        sacc = prefs[0][0] + prefs[0][1]
        for r in prefs[1:]:
            sacc += r[0] + r[1]
        cnt = jnp.maximum(sacc[:, 100:101], 1.0)
        h = jnp.maximum(sacc / cnt, 0.0)
        pooled = jnp.sum(h, axis=0, keepdims=True) * (1.0 / _N)
        z = jnp.dot(pooled, w1h_r[...], preferred_element_type=jnp.float32, precision=lax.Precision.HIGHEST)
        z += jnp.dot(u_ref[...], w1u_r[...], preferred_element_type=jnp.float32, precision=lax.Precision.HIGHEST)
        z = jnp.maximum(z + b1_r[...], 0.0)
        z = jnp.maximum(
            jnp.dot(z, w2_r[...], preferred_element_type=jnp.float32, precision=lax.Precision.HIGHEST) + b2_r[...], 0.0)
        o_ref[...] = jnp.dot(z, w3_r[...], preferred_element_type=jnp.float32, precision=lax.Precision.HIGHEST) + b3_r[...]

    return pl.pallas_call(
        body,
        out_shape=jax.ShapeDtypeStruct((1, 1), jnp.float32),
    )(*([p.reshape(2, _NP, _LP) for p in parts] +
        [u, fw1h, fw1u, fb1, fw2, fb2, fw3, fb3]))


def kernel(x, edge_index, batch, u,
           l0_W1, l0_b1, l0_W2, l0_b2, l0_W3, l0_b3,
           l1_W1, l1_b1, l1_W2, l1_b2, l1_W3, l1_b3,
           f_W1, f_b1, f_W2, f_b2, f_W3, f_b3):
    dst = edge_index[1]
    src = edge_index[0]
    offs = [0]
    for c in _CHUNKS:
        offs.append(offs[-1] + c)
    dsts = tuple(dst[offs[i]:offs[i + 1]] for i in range(len(_CHUNKS)))
    srcs = tuple(src[offs[i]:offs[i + 1]] for i in range(len(_CHUNKS)))

    w1d0 = l0_W1[:_D] - l0_W1[_D:]
    w1s0 = l0_W1[_D:]
    w1d1 = jnp.pad(l1_W1[:_L] - l1_W1[_L:], ((0, _LP - _L), (0, 0)))
    w1s1 = jnp.pad(l1_W1[_L:], ((0, _LP - _L), (0, 0)))
    onehot = jnp.zeros((_LP,), jnp.float32).at[_L].set(1.0)

    def pad3(w3, b3):
        w3p = jnp.pad(w3, ((0, 0), (0, _LP - _L)))
        b3p = jnp.pad(b3, (0, _LP - _L)) + onehot
        return w3p, b3p.reshape(1, _LP)

    w3p0, b3p0 = pad3(l0_W3, l0_b3)
    w3p1, b3p1 = pad3(l1_W3, l1_b3)
    fw1h = jnp.pad(f_W1[:_L], ((0, _LP - _L), (0, 0)))
    fw1u = f_W1[_L:]
    xp = jnp.concatenate([x, jnp.zeros((_N, _LP - _D), x.dtype)], axis=1) \
        if _D != _LP else x

    def split(w):
        bf = jnp.bfloat16
        hi = w.astype(bf)
        lo = (w - hi.astype(jnp.float32)).astype(bf)
        return hi, lo

    def layer(tab, w1d, w1s, b1, w2, b2, w3p, b3p):
        parts = []
        for i, c in enumerate(_CHUNKS):
            gd, gs = _gather_by_size[c](tab, dsts[i], srcs[i])
            m = _edge_mlp(gd, gs, split(w1d), split(w1s), b1,
                          split(w2), b2, split(w3p), b3p)
            parts.append(_scatter_by_size[c](m, dsts[i]))
        return parts

    parts0 = layer(xp, w1d0, w1s0, l0_b1.reshape(1, _H),
                   l0_W2, l0_b2.reshape(1, _H), w3p0, b3p0)
    h0 = _node_mean(parts0)
    parts1 = layer(h0, w1d1, w1s1, l1_b1.reshape(1, _H),
                   l1_W2, l1_b2.reshape(1, _H), w3p1, b3p1)
    return _head(parts1, u, fw1h, fw1u, f_b1.reshape(1, _L),
                 f_W2, f_b2.reshape(1, _L), f_W3, f_b3.reshape(1, 1))


# speedup vs baseline: 2.6984x; 1.0352x over previous
import functools

import jax
import jax.numpy as jnp
from jax import lax
from jax.experimental import pallas as pl
from jax.experimental.pallas import tpu as pltpu
from jax.experimental.pallas import tpu_sc as plsc

_N = 10000
_E = 320000
_CHUNKS = (64000, 64000, 128000, 64000)
_D = 128
_H = 300
_L = 100
_G = 3
_LP = 128
_NW = 32
_NP = 10240
_ROWS_PER_TILE = _NP // 16
_ZR = 64

_CB = 400
_CG = 80
_CS = 80


def _worker_id():
    return lax.axis_index("s") * 2 + lax.axis_index("c")


def _make_gather(eh, dp):
    perw = eh // _NW
    mesh = plsc.VectorSubcoreMesh(core_axis_name="c", subcore_axis_name="s")

    @functools.partial(
        pl.kernel,
        out_type=(jax.ShapeDtypeStruct((eh, dp), jnp.float32),
                  jax.ShapeDtypeStruct((eh, dp), jnp.float32)),
        mesh=mesh,
        scratch_types=[
            pltpu.VMEM((perw,), jnp.int32),
            pltpu.VMEM((perw,), jnp.int32),
            pltpu.VMEM((_CB, dp), jnp.float32),
            pltpu.VMEM((_CB, dp), jnp.float32),
            pltpu.SemaphoreType.DMA,
            pltpu.SemaphoreType.DMA,
            pltpu.SemaphoreType.DMA,
            pltpu.SemaphoreType.DMA,
        ],
    )
    def gk(table, dsti, srci, outd, outs, div, siv, drv, srv, sgd, sgs, swd, sws):
        base = _worker_id() * perw
        pltpu.sync_copy(dsti.at[pl.ds(base, perw)], div)
        pltpu.sync_copy(srci.at[pl.ds(base, perw)], siv)
        nsub = _CB // _CG

        def body(i, carry):
            @pl.when(i > 0)
            def _():
                pltpu.make_async_copy(drv, outd.at[pl.ds(base, _CB)], swd).wait()
                pltpu.make_async_copy(srv, outs.at[pl.ds(base, _CB)], sws).wait()

            cps = []
            for buf, ivec, sem in ((drv, div, sgd), (srv, siv, sgs)):
                for k in range(nsub):
                    o = k * _CG
                    cps.append(pltpu.async_copy(
                        table.at[ivec.at[pl.ds(i * _CB + o, _CG)]],
                        buf.at[pl.ds(o, _CG)], sem))
            for cp in cps[:nsub]:
                cp.wait()
            pltpu.async_copy(drv, outd.at[pl.ds(base + i * _CB, _CB)], swd)
            for cp in cps[nsub:]:
                cp.wait()
            pltpu.async_copy(srv, outs.at[pl.ds(base + i * _CB, _CB)], sws)
            return carry

        lax.fori_loop(0, perw // _CB, body, 0)
        pltpu.make_async_copy(drv, outd.at[pl.ds(base, _CB)], swd).wait()
        pltpu.make_async_copy(srv, outs.at[pl.ds(base, _CB)], sws).wait()

    return gk


_gather_by_size = {eh: _make_gather(eh, _LP) for eh in set(_CHUNKS)}


def _make_scatter(eh):
    perw = eh // _NW
    mesh = plsc.VectorSubcoreMesh(core_axis_name="c", subcore_axis_name="s")

    @functools.partial(
        pl.kernel,
        out_type=jax.ShapeDtypeStruct((2 * _NP, _LP), jnp.float32),
        mesh=mesh,
        scratch_types=[
            pltpu.VMEM((_CS,), jnp.int32),
            pltpu.VMEM((_CS,), jnp.int32),
            pltpu.VMEM((_CS, _LP), jnp.float32),
            pltpu.VMEM((_CS, _LP), jnp.float32),
            pltpu.VMEM((_ZR, _LP), jnp.float32),
            pltpu.VMEM_SHARED((_NP, _LP), jnp.float32),
            pltpu.SemaphoreType.DMA,
            pltpu.SemaphoreType.DMA,
        ],
    )
    def sk(m_hbm, dsti, parts, ib0, ib1, mb0, mb1, zv, acc, sd0, sd1):
        c_ax = lax.axis_index("c")
        s_ax = lax.axis_index("s")
        base = (s_ax * 2 + c_ax) * perw
        ibufs = (ib0, ib1)
        mbufs = (mb0, mb1)
        sds = (sd0, sd1)
        zero16 = jnp.zeros((16,), jnp.float32)

        def zbody(i, carry):
            r = i // (_LP // 16)
            j = i % (_LP // 16)
            zv[r, pl.ds(j * 16, 16)] = zero16
            return carry

        lax.fori_loop(0, _ZR * (_LP // 16), zbody, 0)
        for k in range(_ROWS_PER_TILE // _ZR):
            pltpu.sync_copy(zv, acc.at[pl.ds(s_ax * _ROWS_PER_TILE + k * _ZR, _ZR)])
        plsc.subcore_barrier()

        nchunk = perw // _CS

        def load(p, ci):
            off = base + ci * _CS
            pltpu.async_copy(m_hbm.at[pl.ds(off, _CS)], mbufs[p], sds[p])
            pltpu.async_copy(dsti.at[pl.ds(off, _CS)], ibufs[p], sds[p])

        def drain(p, ci):
            off = base + ci * _CS
            pltpu.make_async_copy(m_hbm.at[pl.ds(off, _CS)], mbufs[p], sds[p]).wait()
            pltpu.make_async_copy(dsti.at[pl.ds(off, _CS)], ibufs[p], sds[p]).wait()

        def scadd(p):
            pltpu.sync_copy(mbufs[p], acc.at[ibufs[p]], add=True)

        npairs = nchunk // 2
        load(0, 0)
        load(1, 1)

        def body(it, carry):
            c = 2 * it
            drain(0, c)
            scadd(0)

            @pl.when(c + 2 < nchunk)
            def _():
                load(0, c + 2)

            drain(1, c + 1)
            scadd(1)

            @pl.when(c + 3 < nchunk)
            def _():
                load(1, c + 3)

            return carry

        lax.fori_loop(0, npairs, body, 0)
        if nchunk % 2:
            drain(0, nchunk - 1)
            scadd(0)
        plsc.subcore_barrier()
        for k in range(_ROWS_PER_TILE // _ZR):
            row0 = s_ax * _ROWS_PER_TILE + k * _ZR
            pltpu.sync_copy(acc.at[pl.ds(row0, _ZR)], zv)
            pltpu.sync_copy(zv, parts.at[pl.ds(c_ax * _NP + row0, _ZR)])

    return sk


_scatter_by_size = {eh: _make_scatter(eh) for eh in set(_CHUNKS)}


def _edge_mlp(gd, gs, w1d, w1s, b1, w2, b2, w3, b3):
    be = 2560
    eh = gd.shape[0]
    din = gd.shape[1]

    bf = jnp.bfloat16

    def dot2(a, w):
        hi = jnp.dot(a, w[0][...], preferred_element_type=jnp.float32)
        lo = jnp.dot(a, w[1][...], preferred_element_type=jnp.float32)
        return hi + lo

    def body(gd_ref, gs_ref, w1dh, w1dl, w1sh, w1sl, b1_r, w2h, w2l, b2_r,
             w3h, w3l, b3_r, o_ref):
        h = dot2(gd_ref[...].astype(bf), (w1dh, w1dl))
        h += dot2(gs_ref[...].astype(bf), (w1sh, w1sl))
        h = jnp.maximum(h + b1_r[...], 0.0)
        h = jnp.maximum(dot2(h.astype(bf), (w2h, w2l)) + b2_r[...], 0.0)
        o_ref[...] = dot2(h.astype(bf), (w3h, w3l)) + b3_r[...]

    full = lambda a: pl.BlockSpec(a.shape, lambda i: (0,) * a.ndim)
    wargs = (w1d[0], w1d[1], w1s[0], w1s[1], b1, w2[0], w2[1], b2,
             w3[0], w3[1], b3)
    return pl.pallas_call(
        body,
        grid=(eh // be,),
        in_specs=[
            pl.BlockSpec((be, din), lambda i: (i, 0)),
            pl.BlockSpec((be, din), lambda i: (i, 0)),
        ] + [full(a) for a in wargs],
        out_specs=pl.BlockSpec((be, _LP), lambda i: (i, 0)),
        out_shape=jax.ShapeDtypeStruct((eh, _LP), jnp.float32),
        compiler_params=pltpu.CompilerParams(
            dimension_semantics=("arbitrary",)),
    )(gd, gs, *wargs)


def _node_mean(parts):
    bn = 2048

    def body(*refs):
        o_ref = refs[-1]
        sacc = refs[0][0] + refs[0][1]
        for r in refs[1:-1]:
            sacc += r[0] + r[1]
        cnt = jnp.maximum(sacc[:, 100:101], 1.0)
        h = jnp.maximum(sacc / cnt, 0.0)
        col = lax.broadcasted_iota(jnp.int32, (bn, _LP), 1)
        o_ref[...] = jnp.where(col < _L, h, 0.0)

    spec = pl.BlockSpec((2, bn, _LP), lambda i: (0, i, 0))
    return pl.pallas_call(
        body,
        grid=(_NP // bn,),
        in_specs=[spec] * len(parts),
        out_specs=pl.BlockSpec((bn, _LP), lambda i: (i, 0)),
        out_shape=jax.ShapeDtypeStruct((_NP, _LP), jnp.float32),
        compiler_params=pltpu.CompilerParams(
            dimension_semantics=("arbitrary",)),
    )(*[p.reshape(2, _NP, _LP) for p in parts])


def _head(parts, u, fw1h, fw1u, fb1, fw2, fb2, fw3, fb3):
    nparts = len(parts)

    def body(*refs):
        prefs = refs[:nparts]
        u_ref, w1h_r, w1u_r, b1_r, w2_r, b2_r, w3_r, b3_r, o_ref = refs[nparts:]
        sacc = prefs[0][0] + prefs[0][1]
        for r in prefs[1:]:
            sacc += r[0] + r[1]
        cnt = jnp.maximum(sacc[:, 100:101], 1.0)
        h = jnp.maximum(sacc / cnt, 0.0)
        pooled = jnp.sum(h, axis=0, keepdims=True) * (1.0 / _N)
        z = jnp.dot(pooled, w1h_r[...], preferred_element_type=jnp.float32, precision=lax.Precision.HIGHEST)
        z += jnp.dot(u_ref[...], w1u_r[...], preferred_element_type=jnp.float32, precision=lax.Precision.HIGHEST)
        z = jnp.maximum(z + b1_r[...], 0.0)
        z = jnp.maximum(
            jnp.dot(z, w2_r[...], preferred_element_type=jnp.float32, precision=lax.Precision.HIGHEST) + b2_r[...], 0.0)
        o_ref[...] = jnp.dot(z, w3_r[...], preferred_element_type=jnp.float32, precision=lax.Precision.HIGHEST) + b3_r[...]

    return pl.pallas_call(
        body,
        out_shape=jax.ShapeDtypeStruct((1, 1), jnp.float32),
    )(*([p.reshape(2, _NP, _LP) for p in parts] +
        [u, fw1h, fw1u, fb1, fw2, fb2, fw3, fb3]))


def kernel(x, edge_index, batch, u,
           l0_W1, l0_b1, l0_W2, l0_b2, l0_W3, l0_b3,
           l1_W1, l1_b1, l1_W2, l1_b2, l1_W3, l1_b3,
           f_W1, f_b1, f_W2, f_b2, f_W3, f_b3):
    dst = edge_index[1]
    src = edge_index[0]
    offs = [0]
    for c in _CHUNKS:
        offs.append(offs[-1] + c)
    dsts = tuple(dst[offs[i]:offs[i + 1]] for i in range(len(_CHUNKS)))
    srcs = tuple(src[offs[i]:offs[i + 1]] for i in range(len(_CHUNKS)))

    w1d0 = l0_W1[:_D] - l0_W1[_D:]
    w1s0 = l0_W1[_D:]
    w1d1 = jnp.pad(l1_W1[:_L] - l1_W1[_L:], ((0, _LP - _L), (0, 0)))
    w1s1 = jnp.pad(l1_W1[_L:], ((0, _LP - _L), (0, 0)))
    onehot = jnp.zeros((_LP,), jnp.float32).at[_L].set(1.0)

    def pad3(w3, b3):
        w3p = jnp.pad(w3, ((0, 0), (0, _LP - _L)))
        b3p = jnp.pad(b3, (0, _LP - _L)) + onehot
        return w3p, b3p.reshape(1, _LP)

    w3p0, b3p0 = pad3(l0_W3, l0_b3)
    w3p1, b3p1 = pad3(l1_W3, l1_b3)
    fw1h = jnp.pad(f_W1[:_L], ((0, _LP - _L), (0, 0)))
    fw1u = f_W1[_L:]
    xp = jnp.concatenate([x, jnp.zeros((_N, _LP - _D), x.dtype)], axis=1) \
        if _D != _LP else x

    def split(w):
        bf = jnp.bfloat16
        hi = w.astype(bf)
        lo = (w - hi.astype(jnp.float32)).astype(bf)
        return hi, lo

    def layer(tab, w1d, w1s, b1, w2, b2, w3p, b3p):
        parts = []
        for i, c in enumerate(_CHUNKS):
            gd, gs = _gather_by_size[c](tab, dsts[i], srcs[i])
            m = _edge_mlp(gd, gs, split(w1d), split(w1s), b1,
                          split(w2), b2, split(w3p), b3p)
            parts.append(_scatter_by_size[c](m, dsts[i]))
        return parts

    parts0 = layer(xp, w1d0, w1s0, l0_b1.reshape(1, _H),
                   l0_W2, l0_b2.reshape(1, _H), w3p0, b3p0)
    h0 = _node_mean(parts0)
    parts1 = layer(h0, w1d1, w1s1, l1_b1.reshape(1, _H),
                   l1_W2, l1_b2.reshape(1, _H), w3p1, b3p1)
    return _head(parts1, u, fw1h, fw1u, f_b1.reshape(1, _L),
                 f_W2, f_b2.reshape(1, _L), f_W3, f_b3.reshape(1, 1))
